# initial kernel scaffold (unmeasured)
import jax
import jax.numpy as jnp
from jax import lax
from jax.experimental import pallas as pl
from jax.experimental.pallas import tpu as pltpu

T = 1024
D = 2048
V_SHARD = 16384
Q_COLS = V_SHARD // 4
TILE = 1024
NT = Q_COLS // TILE


def kernel(x, W, labels):
    labels2d = labels.reshape(T, 1)

    def body(x_ref, w_hbm, lab_ref, out_ref,
             w_buf, comm_send, comm_recv, dma_sems, send_sems, recv_sems,
             exit_sem):
        my_x = lax.axis_index("x")
        my_y = lax.axis_index("y")
        my_z = lax.axis_index("z")
        q = my_y * 2 + my_z
        col0 = q * Q_COLS
        goff = my_x * V_SHARD + col0

        partners = [
            (1 - my_x, my_y, my_z),
            (my_x, 1 - my_y, my_z),
            (my_x, my_y, 1 - my_z),
        ]

        barrier = pltpu.get_barrier_semaphore()
        for p in partners:
            pl.semaphore_signal(barrier, inc=1, device_id=p,
                                device_id_type=pl.DeviceIdType.MESH)
        pl.semaphore_wait(barrier, 3)

        def tile_copy(t, slot):
            return pltpu.make_async_copy(
                w_hbm.at[:, pl.ds(col0 + t * TILE, TILE)],
                w_buf.at[slot],
                dma_sems.at[slot],
            )

        tile_copy(0, 0).start()
        s = jnp.zeros((T, 1), jnp.float32)
        ll = jnp.zeros((T, 1), jnp.float32)
        cols = lax.broadcasted_iota(jnp.int32, (T, TILE), 1)
        for t in range(NT):
            slot = t % 2
            if t + 1 < NT:
                tile_copy(t + 1, (t + 1) % 2).start()
            tile_copy(t, slot).wait()
            logits = jnp.dot(x_ref[...], w_buf[slot],
                             preferred_element_type=jnp.float32)
            s = s + jnp.sum(jnp.exp(logits), axis=1, keepdims=True)
            local = lab_ref[...] - (goff + t * TILE)
            ll = ll + jnp.sum(jnp.where(cols == local, logits, 0.0),
                              axis=1, keepdims=True)

        for r, p in enumerate(partners):
            comm_send[:, 0:1] = s
            comm_send[:, 1:2] = ll
            rdma = pltpu.make_async_remote_copy(
                src_ref=comm_send,
                dst_ref=comm_recv.at[r],
                send_sem=send_sems.at[r],
                recv_sem=recv_sems.at[r],
                device_id=p,
                device_id_type=pl.DeviceIdType.MESH,
            )
            rdma.start()
            rdma.wait()
            s = s + comm_recv[r, :, 0:1]
            ll = ll + comm_recv[r, :, 1:2]

        out_ref[...] = jnp.log(s) - ll

        for p in partners:
            pl.semaphore_signal(exit_sem, inc=1, device_id=p,
                                device_id_type=pl.DeviceIdType.MESH)
        pl.semaphore_wait(exit_sem, 3)

    out = pl.pallas_call(
        body,
        out_shape=jax.ShapeDtypeStruct((T, 1), jnp.float32),
        in_specs=[
            pl.BlockSpec(memory_space=pltpu.VMEM),
            pl.BlockSpec(memory_space=pltpu.ANY),
            pl.BlockSpec(memory_space=pltpu.VMEM),
        ],
        out_specs=pl.BlockSpec(memory_space=pltpu.VMEM),
        scratch_shapes=[
            pltpu.VMEM((2, D, TILE), jnp.float32),
            pltpu.VMEM((T, 2), jnp.float32),
            pltpu.VMEM((3, T, 2), jnp.float32),
            pltpu.SemaphoreType.DMA((2,)),
            pltpu.SemaphoreType.DMA((3,)),
            pltpu.SemaphoreType.DMA((3,)),
            pltpu.SemaphoreType.REGULAR,
        ],
        compiler_params=pltpu.CompilerParams(collective_id=0),
    )(x, W, labels2d)
    return out.reshape(T)


# baseline (device time: 55393 ns/iter reference)
import jax
import jax.numpy as jnp
from jax import lax
from jax.experimental import pallas as pl
from jax.experimental.pallas import tpu as pltpu

T = 1024
D = 2048
V_SHARD = 16384
Q_COLS = V_SHARD // 4
TILE = 1024
NT = Q_COLS // TILE


def kernel(x, W, labels):
    labels2d = labels.reshape(T, 1)

    def body(x_ref, w_hbm, lab_ref, out_ref,
             w_buf, comm_send, comm_recv, dma_sems, send_sems, recv_sems,
             exit_sem):
        my_x = lax.axis_index("x")
        my_y = lax.axis_index("y")
        my_z = lax.axis_index("z")
        q = my_y * 2 + my_z
        col0 = q * Q_COLS
        goff = my_x * V_SHARD + col0

        partners = [
            (1 - my_x, my_y, my_z),
            (my_x, 1 - my_y, my_z),
            (my_x, my_y, 1 - my_z),
        ]

        barrier = pltpu.get_barrier_semaphore()
        for p in partners:
            pl.semaphore_signal(barrier, inc=1, device_id=p,
                                device_id_type=pl.DeviceIdType.MESH)
        pl.semaphore_wait(barrier, 3)

        def tile_copy(t, slot):
            return pltpu.make_async_copy(
                w_hbm.at[:, pl.ds(col0 + t * TILE, TILE)],
                w_buf.at[slot],
                dma_sems.at[slot],
            )

        tile_copy(0, 0).start()
        s = jnp.zeros((T, 1), jnp.float32)
        ll = jnp.zeros((T, 1), jnp.float32)
        cols = lax.broadcasted_iota(jnp.int32, (T, TILE), 1)
        for t in range(NT):
            slot = t % 2
            if t + 1 < NT:
                tile_copy(t + 1, (t + 1) % 2).start()
            tile_copy(t, slot).wait()
            logits = jnp.dot(x_ref[...], w_buf[slot],
                             preferred_element_type=jnp.float32)
            s = s + jnp.sum(jnp.exp(logits), axis=1, keepdims=True)
            local = lab_ref[...] - (goff + t * TILE)
            ll = ll + jnp.sum(jnp.where(cols == local, logits, 0.0),
                              axis=1, keepdims=True)

        for r, p in enumerate(partners):
            comm_send[:, 0:1] = s
            comm_send[:, 1:2] = ll
            rdma = pltpu.make_async_remote_copy(
                src_ref=comm_send,
                dst_ref=comm_recv.at[r],
                send_sem=send_sems.at[r],
                recv_sem=recv_sems.at[r],
                device_id=p,
                device_id_type=pl.DeviceIdType.MESH,
            )
            rdma.start()
            rdma.wait()
            s = s + comm_recv[r, :, 0:1]
            ll = ll + comm_recv[r, :, 1:2]

        out_ref[...] = jnp.log(s) - ll

        for p in partners:
            pl.semaphore_signal(exit_sem, inc=1, device_id=p,
                                device_id_type=pl.DeviceIdType.MESH)
        pl.semaphore_wait(exit_sem, 3)

    out = pl.pallas_call(
        body,
        out_shape=jax.ShapeDtypeStruct((T, 1), jnp.float32),
        in_specs=[
            pl.BlockSpec(memory_space=pltpu.VMEM),
            pl.BlockSpec(memory_space=pl.ANY),
            pl.BlockSpec(memory_space=pltpu.VMEM),
        ],
        out_specs=pl.BlockSpec(memory_space=pltpu.VMEM),
        scratch_shapes=[
            pltpu.VMEM((2, D, TILE), jnp.float32),
            pltpu.VMEM((T, 2), jnp.float32),
            pltpu.VMEM((3, T, 2), jnp.float32),
            pltpu.SemaphoreType.DMA((2,)),
            pltpu.SemaphoreType.DMA((3,)),
            pltpu.SemaphoreType.DMA((3,)),
            pltpu.SemaphoreType.REGULAR,
        ],
        compiler_params=pltpu.CompilerParams(collective_id=0),
    )(x, W, labels2d)
    return out.reshape(T)
